# baseline (speedup 1.0000x reference)
import jax
import jax.numpy as jnp
from jax import lax
from jax.experimental import pallas as pl
from jax.experimental.pallas import tpu as pltpu
from jax.experimental.pallas import tpu_sc as plsc

_B, _H, _W = 8, 512, 512
_RBLK = 256


def _geo_kernel(score_ref, geo_ref, pred_geo_ref, edge_ref, out_ref, acc_ref):
    b = pl.program_id(0)
    i = pl.program_id(1)
    ni = pl.num_programs(1)

    @pl.when(i == 0)
    def _init_batch():
        acc_ref[0] = 0.0
        acc_ref[1] = 0.0

    @pl.when((b == 0) & (i == 0))
    def _init_total():
        acc_ref[2] = 0.0

    score = score_ref[0]
    edge = edge_ref[0]
    x = geo_ref[0] - pred_geo_ref[0]
    y = jnp.clip(x, -1.0, 1.0)
    sl1 = y * (x - 0.5 * y)
    chsum = jnp.sum(sl1, axis=0)
    mask = (score != 0.0).astype(jnp.float32)
    w = mask * (0.125 / edge)

    acc_ref[0] += jnp.sum(mask)
    acc_ref[1] += jnp.sum(chsum * w)

    @pl.when(i == ni - 1)
    def _finish_batch():
        acc_ref[2] += acc_ref[1] / jnp.maximum(acc_ref[0], 1.0)

        @pl.when(b == _B - 1)
        def _finalize():
            out_ref[0] = acc_ref[2] / float(_B)


def _geo_loss_mean(score, geo, pred_geo, edge):
    grid = (_B, _H // _RBLK)
    out = pl.pallas_call(
        _geo_kernel,
        grid=grid,
        in_specs=[
            pl.BlockSpec((1, _RBLK, _W), lambda b, i: (b, i, 0)),
            pl.BlockSpec((1, 8, _RBLK, _W), lambda b, i: (b, 0, i, 0)),
            pl.BlockSpec((1, 8, _RBLK, _W), lambda b, i: (b, 0, i, 0)),
            pl.BlockSpec((1, _RBLK, _W), lambda b, i: (b, i, 0)),
        ],
        out_specs=pl.BlockSpec(memory_space=pltpu.SMEM),
        out_shape=jax.ShapeDtypeStruct((1,), jnp.float32),
        scratch_shapes=[pltpu.SMEM((3,), jnp.float32)],
    )(score, geo, pred_geo, edge)
    return out[0]


_NC, _NS, _L = 2, 16, 16
_NW = _NC * _NS
_ROWS = _B * _H
_RPW = _ROWS // _NW
_CROWS = 32
_NCHUNK = _RPW // _CROWS


_UNROLL = 4


def _dice_partials(score, pred_score):
    s2d = score.reshape(_ROWS, _W)
    p2d = pred_score.reshape(_ROWS, _W)
    mesh = plsc.VectorSubcoreMesh(core_axis_name="c", subcore_axis_name="s")

    def run(s_hbm_arr, p_hbm_arr):
        @pl.kernel(
            out_type=jax.ShapeDtypeStruct((_NW, 2 * _L), jnp.float32),
            mesh=mesh,
            scratch_types=[
                pltpu.VMEM((2, _CROWS, _W), jnp.float32),
                pltpu.VMEM((2, _CROWS, _W), jnp.float32),
                pltpu.VMEM((2 * _L,), jnp.float32),
                pltpu.SemaphoreType.DMA,
                pltpu.SemaphoreType.DMA,
                pltpu.SemaphoreType.DMA,
                pltpu.SemaphoreType.DMA,
            ],
        )
        def k(s_hbm, p_hbm, out_hbm, sbuf, pbuf, obuf, ss0, ss1, sp0, sp1):
            wid = lax.axis_index("s") * _NC + lax.axis_index("c")
            base = wid * _RPW
            sems = ((ss0, sp0), (ss1, sp1))

            def copies(ci, bi):
                row0 = base + ci * _CROWS
                return (
                    pltpu.make_async_copy(
                        s_hbm.at[pl.ds(row0, _CROWS), :], sbuf.at[bi],
                        sems[bi][0]),
                    pltpu.make_async_copy(
                        p_hbm.at[pl.ds(row0, _CROWS), :], pbuf.at[bi],
                        sems[bi][1]),
                )

            for b in range(2):
                for c in copies(b, b):
                    c.start()

            def pair_body(g, accs):
                for b in range(2):
                    ci = g * 2 + b
                    for c in copies(ci, b):
                        c.wait()

                    def row_body(r, accs_r):
                        def vec_body(v, accs2):
                            a2, bc2 = accs2
                            for u in range(_UNROLL):
                                off = (v * _UNROLL + u) * _L
                                s = sbuf[b, r, pl.ds(off, _L)]
                                p = pbuf[b, r, pl.ds(off, _L)]
                                a2 = a2 + s * p
                                bc2 = bc2 + (s + p)
                            return (a2, bc2)

                        return lax.fori_loop(0, _W // (_L * _UNROLL),
                                             vec_body, accs_r)

                    accs = lax.fori_loop(0, _CROWS, row_body, accs)

                    nxt = ci + 2

                    @pl.when(nxt < _NCHUNK)
                    def _prefetch():
                        for c in copies(nxt, b):
                            c.start()

                return accs

            zero = jnp.zeros((_L,), jnp.float32)
            acc_a, acc_bc = lax.fori_loop(0, _NCHUNK // 2, pair_body,
                                          (zero, zero))
            obuf[pl.ds(0, _L)] = acc_a
            obuf[pl.ds(_L, _L)] = acc_bc
            pltpu.sync_copy(obuf, out_hbm.at[wid])

        return k(s_hbm_arr, p_hbm_arr)

    return run(s2d, p2d)


def kernel(score, pred_score, geo, pred_geo, edge):
    partials = _dice_partials(score, pred_score)
    geo_mean = _geo_loss_mean(score, geo, pred_geo, edge)
    per_batch = partials.reshape(_B, _NW // _B, 2, _L).sum(axis=(1, 3))
    a, bc = per_batch[:, 0], per_batch[:, 1]
    dice = 1.0 - 2.0 * a / bc
    return jnp.mean(dice) + geo_mean


# speedup vs baseline: 1.4176x; 1.4176x over previous
import jax
import jax.numpy as jnp
from jax.experimental import pallas as pl
from jax.experimental.pallas import tpu as pltpu

_B, _H, _W = 8, 512, 512
_RBLK = 256


def _loss_kernel(score_ref, pred_score_ref, geo_ref, pred_geo_ref, edge_ref,
                 out_ref, acc_ref):
    b = pl.program_id(0)
    i = pl.program_id(1)
    ni = pl.num_programs(1)

    @pl.when(i == 0)
    def _init_batch():
        for k in range(5):
            acc_ref[k] = 0.0

    @pl.when((b == 0) & (i == 0))
    def _init_total():
        acc_ref[5] = 0.0

    score = score_ref[0]
    ps = pred_score_ref[0]
    edge = edge_ref[0]
    x = geo_ref[0] - pred_geo_ref[0]
    y = jnp.clip(x, -1.0, 1.0)
    sl1 = y * (x - 0.5 * y)
    chsum = jnp.sum(sl1, axis=0)
    mask = (score != 0.0).astype(jnp.float32)
    w = mask * (0.125 / edge)

    acc_ref[0] += jnp.sum(score * ps)
    acc_ref[1] += jnp.sum(score)
    acc_ref[2] += jnp.sum(ps)
    acc_ref[3] += jnp.sum(mask)
    acc_ref[4] += jnp.sum(chsum * w)

    @pl.when(i == ni - 1)
    def _finish_batch():
        a, bs, c, dn, e = (acc_ref[0], acc_ref[1], acc_ref[2],
                           acc_ref[3], acc_ref[4])
        dice = 1.0 - 2.0 * a / (bs + c)
        geo_loss = e / jnp.maximum(dn, 1.0)
        acc_ref[5] += dice + geo_loss

        @pl.when(b == _B - 1)
        def _finalize():
            out_ref[0] = acc_ref[5] / float(_B)


def kernel(score, pred_score, geo, pred_geo, edge):
    grid = (_B, _H // _RBLK)
    out = pl.pallas_call(
        _loss_kernel,
        grid=grid,
        in_specs=[
            pl.BlockSpec((1, _RBLK, _W), lambda b, i: (b, i, 0)),
            pl.BlockSpec((1, _RBLK, _W), lambda b, i: (b, i, 0)),
            pl.BlockSpec((1, 8, _RBLK, _W), lambda b, i: (b, 0, i, 0)),
            pl.BlockSpec((1, 8, _RBLK, _W), lambda b, i: (b, 0, i, 0)),
            pl.BlockSpec((1, _RBLK, _W), lambda b, i: (b, i, 0)),
        ],
        out_specs=pl.BlockSpec(memory_space=pltpu.SMEM),
        out_shape=jax.ShapeDtypeStruct((1,), jnp.float32),
        scratch_shapes=[pltpu.SMEM((6,), jnp.float32)],
    )(score, pred_score, geo, pred_geo, edge)
    return out[0]
